# baseline (speedup 1.0000x reference)
import functools

import jax
import jax.numpy as jnp
from jax import lax
from jax.experimental import pallas as pl
from jax.experimental.pallas import tpu as pltpu
from jax.experimental.pallas import tpu_sc as plsc

N_EDGES = 320000
N_SEGMENTS = 10000
D = 128

TCB = 1024
NB_TC = 140
E_SPLIT = NB_TC * TCB
SW = 128
NWIN = 80
NPAIR = NB_TC + 88
TC_ROWS = (NWIN + 1) * SW

NUM_CORES = 2
NUM_SUBCORES = 16
NUM_WORKERS = NUM_CORES * NUM_SUBCORES

CHUNK = 128
N_CHUNKS = N_EDGES // CHUNK
CH0 = E_SPLIT // CHUNK
N_SC_CHUNKS = N_CHUNKS - CH0
BASE_CPW = N_SC_CHUNKS // NUM_WORKERS
EXTRA = N_SC_CHUNKS - BASE_CPW * NUM_WORKERS
MAX_CPW = BASE_CPW + 1
IDS_PAD = 2504
IDS_ENV = 56
ENV_LIM = IDS_PAD - IDS_ENV

SEG_PAD = 10240
ROWS_PER_TILE = SEG_PAD // NUM_SUBCORES
WCHUNK = 128
N_WCHUNKS = ROWS_PER_TILE // WCHUNK

NBUF = 2
N_SLOTS = 44
N_GROUPS = N_SLOTS // NBUF


def _sc_segment_sum(x, ids2, zrows):
    mesh = plsc.VectorSubcoreMesh(
        core_axis_name="c", subcore_axis_name="s",
        num_cores=NUM_CORES, num_subcores=NUM_SUBCORES)

    @functools.partial(
        pl.kernel,
        out_type=jax.ShapeDtypeStruct((NUM_CORES, SEG_PAD, D), jnp.float32),
        mesh=mesh,
        scratch_types=[
            [pltpu.VMEM((CHUNK, D), jnp.float32) for _ in range(NBUF)],
            [pltpu.SemaphoreType.DMA for _ in range(NBUF)],
            pltpu.VMEM((IDS_ENV, CHUNK), jnp.int32),
            pltpu.VMEM_SHARED((SEG_PAD, D), jnp.float32),
        ],
    )
    def kern(x_hbm, ids_hbm, z_hbm, out_hbm, xbufs, sems, idbuf, pooled):
        c = lax.axis_index("c")
        s = lax.axis_index("s")
        wid = c * NUM_SUBCORES + s

        pltpu.sync_copy(z_hbm, xbufs[0])
        base_row = s * ROWS_PER_TILE

        def zero_body(j, _):
            pltpu.sync_copy(xbufs[0],
                            pooled.at[pl.ds(base_row + j * WCHUNK, WCHUNK)])
            return 0

        lax.fori_loop(0, N_WCHUNKS, zero_body, 0)

        start = CH0 + wid * BASE_CPW + jnp.minimum(wid, EXTRA)
        count = BASE_CPW + jnp.where(wid < EXTRA, 1, 0)

        start_al = pl.multiple_of(
            jnp.minimum((start // 8) * 8, ENV_LIM), 8)
        off = start - start_al
        pltpu.sync_copy(ids_hbm.at[pl.ds(start_al, IDS_ENV)], idbuf)
        plsc.subcore_barrier()

        def fill(j, b):
            return pltpu.make_async_copy(
                x_hbm.at[pl.ds((start + j) * CHUNK, CHUNK)], xbufs[b], sems[b])

        for b in range(NBUF):
            fill(b, b).start()

        @pl.loop(0, N_GROUPS)
        def g_loop(g):
            for b in range(NBUF):
                j = g * NBUF + b

                @pl.when(j < count)
                def _():
                    fill(j, b).wait()
                    pltpu.sync_copy(xbufs[b], pooled.at[idbuf.at[off + j]],
                                    add=True)

                    @pl.when(j + NBUF < count)
                    def _():
                        fill(j + NBUF, b).start()

        plsc.subcore_barrier()

        def out_body(j, _):
            r = base_row + j * WCHUNK
            pltpu.sync_copy(pooled.at[pl.ds(r, WCHUNK)],
                            out_hbm.at[c, pl.ds(r, WCHUNK)])
            return 0

        lax.fori_loop(0, N_WCHUNKS, out_body, 0)

    return kern(x, ids2, zrows)


def _tc_onehot_body(kref, wref, fref, ids_ref, x_ref, o_ref):
    i = pl.program_id(0)
    wb = wref[i] * SW
    ids_blk = ids_ref[0, 0, :]
    seg = wb + lax.broadcasted_iota(jnp.int32, (SW, TCB), 0)
    oh = (seg == ids_blk[None, :]).astype(jnp.float32)
    acc = lax.dot_general(oh, x_ref[...], (((1,), (0,)), ((), ())),
                          preferred_element_type=jnp.float32,
                          precision=lax.Precision.HIGHEST)

    @pl.when(fref[i] == 1)
    def _():
        o_ref[...] = acc

    @pl.when(fref[i] == 0)
    def _():
        o_ref[...] += acc


def _tc_onehot(x, ids3, blk_of, win_of, first_of):
    grid_spec = pltpu.PrefetchScalarGridSpec(
        num_scalar_prefetch=3,
        grid=(NPAIR,),
        in_specs=[
            pl.BlockSpec((1, 1, TCB), lambda i, k, w, f: (k[i], 0, 0)),
            pl.BlockSpec((TCB, D), lambda i, k, w, f: (k[i], 0)),
        ],
        out_specs=pl.BlockSpec((SW, D), lambda i, k, w, f: (w[i], 0)),
    )
    return pl.pallas_call(
        _tc_onehot_body,
        grid_spec=grid_spec,
        out_shape=jax.ShapeDtypeStruct((TC_ROWS, D), jnp.float32),
    )(blk_of, win_of, first_of, ids3, x)


BLK = 2000


def _tc_linear_body(bnd_ref, p_ref, t_ref, w_ref, b_ref, o_ref):
    i = pl.program_id(0)
    row = i * BLK + lax.broadcasted_iota(jnp.int32, (BLK, D), 0)
    valid = (row >= bnd_ref[0]) & (row < bnd_ref[1])
    p = p_ref[0] + p_ref[1] + jnp.where(valid, t_ref[...], 0.0)
    o_ref[...] = lax.dot_general(
        p, w_ref[...], (((1,), (1,)), ((), ())),
        preferred_element_type=jnp.float32) + b_ref[...]


def _tc_linear(bounds, partials, tc_pooled, W, b):
    grid_spec = pltpu.PrefetchScalarGridSpec(
        num_scalar_prefetch=1,
        grid=(N_SEGMENTS // BLK,),
        in_specs=[
            pl.BlockSpec((NUM_CORES, BLK, D), lambda i, bnd: (0, i, 0)),
            pl.BlockSpec((BLK, D), lambda i, bnd: (i, 0)),
            pl.BlockSpec((D, D), lambda i, bnd: (0, 0)),
            pl.BlockSpec((1, D), lambda i, bnd: (0, 0)),
        ],
        out_specs=pl.BlockSpec((BLK, D), lambda i, bnd: (i, 0)),
    )
    return pl.pallas_call(
        _tc_linear_body,
        grid_spec=grid_spec,
        out_shape=jax.ShapeDtypeStruct((N_SEGMENTS, D), jnp.float32),
    )(bounds, partials, tc_pooled, W, b)


@jax.jit
def kernel(x, batch, W, b):
    batch32 = batch.astype(jnp.int32)
    ids2 = jnp.pad(batch32.reshape(N_CHUNKS, CHUNK),
                   ((0, IDS_PAD - N_CHUNKS), (0, 0)))
    zrows = jnp.zeros((WCHUNK, D), jnp.float32)

    ids_tc = batch32[:E_SPLIT].reshape(NB_TC, TCB)
    w_lo = ids_tc[:, 0] // SW
    w_hi = ids_tc[:, -1] // SW
    w_hi_eff = jnp.maximum(w_hi, jnp.concatenate([w_lo[1:], w_hi[-1:]]) - 1)
    spans = w_hi_eff - w_lo + 1
    offs = jnp.concatenate([jnp.zeros((1,), jnp.int32),
                            jnp.cumsum(spans, dtype=jnp.int32)])
    total = offs[-1]
    i = jnp.arange(NPAIR, dtype=jnp.int32)
    k = jnp.clip(jnp.searchsorted(offs, i, side='right').astype(jnp.int32) - 1,
                 0, NB_TC - 1)
    win = jnp.where(i < total, w_lo[k] + (i - offs[k]), NWIN)
    first = jnp.concatenate([jnp.ones((1,), jnp.int32),
                             (win[1:] != win[:-1]).astype(jnp.int32)])
    bounds = jnp.stack([w_lo[0] * SW, (w_hi[-1] + 1) * SW]).astype(jnp.int32)

    partials = _sc_segment_sum(x, ids2, zrows)
    tc_pooled = _tc_onehot(x, ids_tc.reshape(NB_TC, 1, TCB), k, win, first)
    return _tc_linear(bounds, partials, tc_pooled, W, b.reshape(1, D))


# speedup vs baseline: 1.8102x; 1.8102x over previous
import functools

import jax
import jax.numpy as jnp
from jax import lax
from jax.experimental import pallas as pl
from jax.experimental.pallas import tpu as pltpu
from jax.experimental.pallas import tpu_sc as plsc

N_EDGES = 320000
N_SEGMENTS = 10000
D = 128

NUM_CORES = 2
NUM_SUBCORES = 16
NUM_WORKERS = NUM_CORES * NUM_SUBCORES

CHUNK = 128
N_CHUNKS = N_EDGES // CHUNK
BASE_CPW = N_CHUNKS // NUM_WORKERS
EXTRA = N_CHUNKS - BASE_CPW * NUM_WORKERS
MAX_CPW = BASE_CPW + 1
IDS_ENV = 88
IDS_PAD = 2504
ENV_LIM = IDS_PAD - IDS_ENV

SEG_PAD = 10240
ROWS_PER_TILE = SEG_PAD // NUM_SUBCORES
WCHUNK = 128
N_WCHUNKS = ROWS_PER_TILE // WCHUNK

NBUF = 2
N_SLOTS = 80
N_GROUPS = N_SLOTS // NBUF


def _sc_segment_sum(x, ids2, zrows):
    mesh = plsc.VectorSubcoreMesh(
        core_axis_name="c", subcore_axis_name="s",
        num_cores=NUM_CORES, num_subcores=NUM_SUBCORES)

    @functools.partial(
        pl.kernel,
        out_type=jax.ShapeDtypeStruct((NUM_CORES, SEG_PAD, D), jnp.float32),
        mesh=mesh,
        scratch_types=[
            [pltpu.VMEM((CHUNK, D), jnp.float32) for _ in range(NBUF)],
            [pltpu.SemaphoreType.DMA for _ in range(NBUF)],
            pltpu.VMEM((IDS_ENV, CHUNK), jnp.int32),
            pltpu.VMEM_SHARED((SEG_PAD, D), jnp.float32),
        ],
    )
    def kern(x_hbm, ids_hbm, z_hbm, out_hbm, xbufs, sems, idbuf, pooled):
        c = lax.axis_index("c")
        s = lax.axis_index("s")
        wid = c * NUM_SUBCORES + s

        pltpu.sync_copy(z_hbm, xbufs[0])
        base_row = s * ROWS_PER_TILE

        def zero_body(j, _):
            pltpu.sync_copy(xbufs[0],
                            pooled.at[pl.ds(base_row + j * WCHUNK, WCHUNK)])
            return 0

        lax.fori_loop(0, N_WCHUNKS, zero_body, 0)

        start = wid * BASE_CPW + jnp.minimum(wid, EXTRA)
        count = BASE_CPW + jnp.where(wid < EXTRA, 1, 0)

        start_al = pl.multiple_of(
            jnp.minimum((start // 8) * 8, ENV_LIM), 8)
        off = start - start_al
        pltpu.sync_copy(ids_hbm.at[pl.ds(start_al, IDS_ENV)], idbuf)
        plsc.subcore_barrier()

        def fill(j, b):
            return pltpu.make_async_copy(
                x_hbm.at[pl.ds((start + j) * CHUNK, CHUNK)], xbufs[b], sems[b])

        for b in range(NBUF):
            fill(b, b).start()

        @pl.loop(0, N_GROUPS)
        def g_loop(g):
            for b in range(NBUF):
                j = g * NBUF + b

                @pl.when(j < count)
                def _():
                    fill(j, b).wait()
                    pltpu.sync_copy(xbufs[b], pooled.at[idbuf.at[off + j]],
                                    add=True)

                    @pl.when(j + NBUF < count)
                    def _():
                        fill(j + NBUF, b).start()

        plsc.subcore_barrier()

        def out_body(j, _):
            r = base_row + j * WCHUNK
            pltpu.sync_copy(pooled.at[pl.ds(r, WCHUNK)],
                            out_hbm.at[c, pl.ds(r, WCHUNK)])
            return 0

        lax.fori_loop(0, N_WCHUNKS, out_body, 0)

    return kern(x, ids2, zrows)


BLK = 2000


def _tc_linear_body(p_ref, w_ref, b_ref, o_ref):
    p = p_ref[0] + p_ref[1]
    o_ref[...] = lax.dot_general(
        p, w_ref[...], (((1,), (1,)), ((), ())),
        preferred_element_type=jnp.float32) + b_ref[...]


def _tc_linear(partials, W, b):
    return pl.pallas_call(
        _tc_linear_body,
        grid=(N_SEGMENTS // BLK,),
        in_specs=[
            pl.BlockSpec((NUM_CORES, BLK, D), lambda i: (0, i, 0)),
            pl.BlockSpec((D, D), lambda i: (0, 0)),
            pl.BlockSpec((1, D), lambda i: (0, 0)),
        ],
        out_specs=pl.BlockSpec((BLK, D), lambda i: (i, 0)),
        out_shape=jax.ShapeDtypeStruct((N_SEGMENTS, D), jnp.float32),
    )(partials, W, b)


@jax.jit
def kernel(x, batch, W, b):
    ids2 = batch.astype(jnp.int32).reshape(N_CHUNKS, CHUNK)
    ids2 = jnp.pad(ids2, ((0, IDS_PAD - N_CHUNKS), (0, 0)))
    zrows = jnp.zeros((WCHUNK, D), jnp.float32)
    partials = _sc_segment_sum(x, ids2, zrows)
    return _tc_linear(partials, W, b.reshape(1, D))
